# baseline (speedup 1.0000x reference)
import functools

import jax
import jax.numpy as jnp
from jax import lax
from jax.experimental import pallas as pl
from jax.experimental.pallas import tpu as pltpu
from jax.experimental.pallas import tpu_sc as plsc

N_ROWS = 1000000
EMB_DIM = 32
PAIRS = EMB_DIM // 2
LANES = 16
PANELS = 8192
DIM_STRIDE = PANELS * 128
BLK_COLS = 32768


def _pack_block(x):
    bits = jax.lax.bitcast_convert_type(x, jnp.uint32)
    lo = bits[:PAIRS, :] >> 16
    hi = bits[PAIRS:, :] & jnp.uint32(0xFFFF0000)
    packed = lo | hi
    return packed.reshape(PAIRS, BLK_COLS // 128, 128)


def _detile_body(t_in, t_out):
    t_out[...] = _pack_block(t_in[...])


def _detile(tt):
    grid = (N_ROWS + BLK_COLS - 1) // BLK_COLS
    out = pl.pallas_call(
        _detile_body,
        grid=(grid,),
        in_specs=[pl.BlockSpec((EMB_DIM, BLK_COLS), lambda j: (0, j))],
        out_specs=pl.BlockSpec((PAIRS, BLK_COLS // 128, 128),
                               lambda j: (0, j, 0)),
        out_shape=jax.ShapeDtypeStruct((PAIRS, PANELS, 128), jnp.uint32),
    )(tt)
    return out.reshape(-1)


def _common(batch):
    info = plsc.get_sparse_core_info()
    nc, ns = info.num_cores, info.num_subcores
    nw = nc * ns
    assert batch % (8 * nw) == 0
    return nc, batch // nw


def _start_gathers(flat_hbm, idx_v, dat_v, bias_hbm, bias_out_v, sem,
                   b_per_w):
    copies = []
    for d in range(PAIRS):
        cp = pltpu.make_async_copy(
            flat_hbm.at[pl.ds(d * DIM_STRIDE, DIM_STRIDE)].at[idx_v],
            dat_v.at[pl.ds(d * b_per_w, b_per_w)], sem)
        cp.start()
        copies.append(cp)
    cp = pltpu.make_async_copy(bias_hbm.at[idx_v], bias_out_v, sem)
    cp.start()
    copies.append(cp)
    return copies


def _make_gather_stage(batch):
    nc, b_per_w = _common(batch)
    mesh = plsc.VectorSubcoreMesh(core_axis_name="c", subcore_axis_name="s")

    @functools.partial(
        pl.kernel,
        mesh=mesh,
        out_type=(
            jax.ShapeDtypeStruct((PAIRS * batch,), jnp.uint32),
            jax.ShapeDtypeStruct((batch,), jnp.float32),
        ),
        scratch_types=[
            pltpu.VMEM((b_per_w,), jnp.int32),
            pltpu.VMEM((PAIRS * b_per_w,), jnp.uint32),
            pltpu.VMEM((b_per_w,), jnp.float32),
            pltpu.SemaphoreType.DMA,
        ],
        compiler_params=pltpu.CompilerParams(needs_layout_passes=False),
    )
    def k(users_hbm, uflat_hbm, ubias_hbm, ug_hbm, ubg_hbm,
          users_v, udat_v, ubias_v, sem):
        wid = lax.axis_index("s") * nc + lax.axis_index("c")
        base = wid * b_per_w

        pltpu.sync_copy(users_hbm.at[pl.ds(base, b_per_w)], users_v)
        for cp in _start_gathers(uflat_hbm, users_v, udat_v, ubias_hbm,
                                 ubias_v, sem, b_per_w):
            cp.wait()
        pltpu.sync_copy(udat_v,
                        ug_hbm.at[pl.ds(wid * PAIRS * b_per_w,
                                        PAIRS * b_per_w)])
        pltpu.sync_copy(ubias_v, ubg_hbm.at[pl.ds(base, b_per_w)])

    return k


def _make_finish_stage(batch):
    nc, b_per_w = _common(batch)
    n_groups = b_per_w // LANES
    mesh = plsc.VectorSubcoreMesh(core_axis_name="c", subcore_axis_name="s")

    @functools.partial(
        pl.kernel,
        mesh=mesh,
        out_type=jax.ShapeDtypeStruct((batch,), jnp.float32),
        scratch_types=[
            pltpu.VMEM((b_per_w,), jnp.int32),
            pltpu.VMEM((PAIRS * b_per_w,), jnp.uint32),
            pltpu.VMEM((PAIRS * b_per_w,), jnp.uint32),
            pltpu.VMEM((b_per_w,), jnp.float32),
            pltpu.VMEM((b_per_w,), jnp.float32),
            pltpu.VMEM((LANES,), jnp.float32),
            pltpu.VMEM((b_per_w,), jnp.float32),
            pltpu.SemaphoreType.DMA,
        ],
        compiler_params=pltpu.CompilerParams(needs_layout_passes=False),
    )
    def k(items_hbm, iflat_hbm, ibias_hbm, ug_hbm, ubg_hbm, bias_hbm,
          out_hbm, items_v, idat_v, udat_v, ibias_v, ubias_v, bias_v,
          out_v, sem):
        wid = lax.axis_index("s") * nc + lax.axis_index("c")
        base = wid * b_per_w

        pltpu.sync_copy(items_hbm.at[pl.ds(base, b_per_w)], items_v)
        copies = _start_gathers(iflat_hbm, items_v, idat_v, ibias_hbm,
                                ibias_v, sem, b_per_w)
        pltpu.sync_copy(ug_hbm.at[pl.ds(wid * PAIRS * b_per_w,
                                        PAIRS * b_per_w)], udat_v)
        pltpu.sync_copy(ubg_hbm.at[pl.ds(base, b_per_w)], ubias_v)
        pltpu.sync_copy(bias_hbm, bias_v)
        for cp in copies:
            cp.wait()

        bias_vec = bias_v[...]
        himask = jnp.full((LANES,), 0xFFFF0000, jnp.uint32)

        def split(w):
            lo = plsc.bitcast(w << 16, jnp.float32)
            hi = plsc.bitcast(w & himask, jnp.float32)
            return lo, hi

        def group(g, carry):
            e0 = g * LANES
            acc = jnp.zeros((LANES,), jnp.float32)
            for d in range(PAIRS):
                sl = pl.ds(d * b_per_w + e0, LANES)
                ulo, uhi = split(udat_v[sl])
                ilo, ihi = split(idat_v[sl])
                acc = acc + ulo * ilo + uhi * ihi
            sl = pl.ds(e0, LANES)
            acc = acc + ubias_v[sl] + ibias_v[sl] + bias_vec
            out_v[sl] = 1.0 / (1.0 + jnp.exp(-acc))
            return carry

        lax.fori_loop(0, n_groups, group, 0)
        pltpu.sync_copy(out_v, out_hbm.at[pl.ds(base, b_per_w)])

    return k


@jax.jit
def kernel(users, items, user_table, item_table, user_bias, item_bias, bias):
    batch = users.shape[0]
    users = users.astype(jnp.int32)
    items = items.astype(jnp.int32)
    bias16 = jnp.broadcast_to(bias.astype(jnp.float32), (LANES,))
    uflat = _detile(user_table.T)
    ug, ubg = _make_gather_stage(batch)(users, uflat, user_bias)
    iflat = _detile(item_table.T)
    return _make_finish_stage(batch)(items, iflat, item_bias, ug, ubg,
                                     bias16)


# speedup vs baseline: 1.0188x; 1.0188x over previous
import functools

import jax
import jax.numpy as jnp
from jax import lax
from jax.experimental import pallas as pl
from jax.experimental.pallas import tpu as pltpu
from jax.experimental.pallas import tpu_sc as plsc

N_ROWS = 1000000
EMB_DIM = 32
PAIRS = EMB_DIM // 2
LANES = 16
PANELS = 8192
DIM_STRIDE = PANELS * 128
BLK_COLS = 65536


def _pack_block(x):
    bits = jax.lax.bitcast_convert_type(x, jnp.uint32)
    lo = bits[:PAIRS, :] >> 16
    hi = bits[PAIRS:, :] & jnp.uint32(0xFFFF0000)
    packed = lo | hi
    return packed.reshape(PAIRS, BLK_COLS // 128, 128)


def _detile_body(t_in, t_out):
    t_out[...] = _pack_block(t_in[...])


def _detile(tt):
    grid = (N_ROWS + BLK_COLS - 1) // BLK_COLS
    out = pl.pallas_call(
        _detile_body,
        grid=(grid,),
        in_specs=[pl.BlockSpec((EMB_DIM, BLK_COLS), lambda j: (0, j))],
        out_specs=pl.BlockSpec((PAIRS, BLK_COLS // 128, 128),
                               lambda j: (0, j, 0)),
        out_shape=jax.ShapeDtypeStruct((PAIRS, PANELS, 128), jnp.uint32),
    )(tt)
    return out.reshape(-1)


def _common(batch):
    info = plsc.get_sparse_core_info()
    nc, ns = info.num_cores, info.num_subcores
    nw = nc * ns
    assert batch % (8 * nw) == 0
    return nc, batch // nw


def _start_gathers(flat_hbm, idx_v, dat_v, bias_hbm, bias_out_v, sem,
                   b_per_w):
    copies = []
    for d in range(PAIRS):
        cp = pltpu.make_async_copy(
            flat_hbm.at[pl.ds(d * DIM_STRIDE, DIM_STRIDE)].at[idx_v],
            dat_v.at[pl.ds(d * b_per_w, b_per_w)], sem)
        cp.start()
        copies.append(cp)
    cp = pltpu.make_async_copy(bias_hbm.at[idx_v], bias_out_v, sem)
    cp.start()
    copies.append(cp)
    return copies


def _make_gather_stage(batch):
    nc, b_per_w = _common(batch)
    mesh = plsc.VectorSubcoreMesh(core_axis_name="c", subcore_axis_name="s")

    @functools.partial(
        pl.kernel,
        mesh=mesh,
        out_type=(
            jax.ShapeDtypeStruct((PAIRS * batch,), jnp.uint32),
            jax.ShapeDtypeStruct((batch,), jnp.float32),
        ),
        scratch_types=[
            pltpu.VMEM((b_per_w,), jnp.int32),
            pltpu.VMEM((PAIRS * b_per_w,), jnp.uint32),
            pltpu.VMEM((b_per_w,), jnp.float32),
            pltpu.SemaphoreType.DMA,
        ],
        compiler_params=pltpu.CompilerParams(needs_layout_passes=False),
    )
    def k(users_hbm, uflat_hbm, ubias_hbm, ug_hbm, ubg_hbm,
          users_v, udat_v, ubias_v, sem):
        wid = lax.axis_index("s") * nc + lax.axis_index("c")
        base = wid * b_per_w

        pltpu.sync_copy(users_hbm.at[pl.ds(base, b_per_w)], users_v)
        for cp in _start_gathers(uflat_hbm, users_v, udat_v, ubias_hbm,
                                 ubias_v, sem, b_per_w):
            cp.wait()
        pltpu.sync_copy(udat_v,
                        ug_hbm.at[pl.ds(wid * PAIRS * b_per_w,
                                        PAIRS * b_per_w)])
        pltpu.sync_copy(ubias_v, ubg_hbm.at[pl.ds(base, b_per_w)])

    return k


def _make_finish_stage(batch):
    nc, b_per_w = _common(batch)
    n_groups = b_per_w // LANES
    mesh = plsc.VectorSubcoreMesh(core_axis_name="c", subcore_axis_name="s")

    @functools.partial(
        pl.kernel,
        mesh=mesh,
        out_type=jax.ShapeDtypeStruct((batch,), jnp.float32),
        scratch_types=[
            pltpu.VMEM((b_per_w,), jnp.int32),
            pltpu.VMEM((PAIRS * b_per_w,), jnp.uint32),
            pltpu.VMEM((PAIRS * b_per_w,), jnp.uint32),
            pltpu.VMEM((b_per_w,), jnp.float32),
            pltpu.VMEM((b_per_w,), jnp.float32),
            pltpu.VMEM((LANES,), jnp.float32),
            pltpu.VMEM((b_per_w,), jnp.float32),
            pltpu.SemaphoreType.DMA,
        ],
        compiler_params=pltpu.CompilerParams(needs_layout_passes=False),
    )
    def k(items_hbm, iflat_hbm, ibias_hbm, ug_hbm, ubg_hbm, bias_hbm,
          out_hbm, items_v, idat_v, udat_v, ibias_v, ubias_v, bias_v,
          out_v, sem):
        wid = lax.axis_index("s") * nc + lax.axis_index("c")
        base = wid * b_per_w

        pltpu.sync_copy(items_hbm.at[pl.ds(base, b_per_w)], items_v)
        copies = _start_gathers(iflat_hbm, items_v, idat_v, ibias_hbm,
                                ibias_v, sem, b_per_w)
        pltpu.sync_copy(ug_hbm.at[pl.ds(wid * PAIRS * b_per_w,
                                        PAIRS * b_per_w)], udat_v)
        pltpu.sync_copy(ubg_hbm.at[pl.ds(base, b_per_w)], ubias_v)
        pltpu.sync_copy(bias_hbm, bias_v)
        for cp in copies:
            cp.wait()

        bias_vec = bias_v[...]
        himask = jnp.full((LANES,), 0xFFFF0000, jnp.uint32)

        def split(w):
            lo = plsc.bitcast(w << 16, jnp.float32)
            hi = plsc.bitcast(w & himask, jnp.float32)
            return lo, hi

        def group(g, carry):
            e0 = g * LANES
            acc = jnp.zeros((LANES,), jnp.float32)
            for d in range(PAIRS):
                sl = pl.ds(d * b_per_w + e0, LANES)
                ulo, uhi = split(udat_v[sl])
                ilo, ihi = split(idat_v[sl])
                acc = acc + ulo * ilo + uhi * ihi
            sl = pl.ds(e0, LANES)
            acc = acc + ubias_v[sl] + ibias_v[sl] + bias_vec
            out_v[sl] = 1.0 / (1.0 + jnp.exp(-acc))
            return carry

        lax.fori_loop(0, n_groups, group, 0)
        pltpu.sync_copy(out_v, out_hbm.at[pl.ds(base, b_per_w)])

    return k


@jax.jit
def kernel(users, items, user_table, item_table, user_bias, item_bias, bias):
    batch = users.shape[0]
    users = users.astype(jnp.int32)
    items = items.astype(jnp.int32)
    bias16 = jnp.broadcast_to(bias.astype(jnp.float32), (LANES,))
    uflat = _detile(user_table.T)
    ug, ubg = _make_gather_stage(batch)(users, uflat, user_bias)
    iflat = _detile(item_table.T)
    return _make_finish_stage(batch)(items, iflat, item_bias, ug, ubg,
                                     bias16)
